# baseline (speedup 1.0000x reference)
import jax
import jax.numpy as jnp
from jax import lax
from jax.experimental import pallas as pl
from jax.experimental.pallas import tpu as pltpu
from jax.experimental.pallas import tpu_sc as plsc

N = 10000
E = 320000
D = 128
NC = 2
NS = 16
DH = D // NC
EP = E // NS
NH = 2
EH = EP // NH
C = 80
NCH = EH // C
NBUF = 5
NSB = 4
SB_MAP = (0, 1, 2, 3, 0)
SB_PREV = (1, 5, 5, 5, 4)
ACC_N = 10240
RPT = ACC_N // NS
ZB = RPT // C


def _mm_body(h_ref, w_ref, o_ref):
    o_ref[...] = jnp.dot(h_ref[...], w_ref[...],
                         preferred_element_type=jnp.float32)


def _combine_body(p_ref, b_ref, o_ref):
    o_ref[...] = jnp.concatenate([p_ref[0], p_ref[1]], axis=1) + b_ref[...]


def _sc_body(hw_hbm, norm_hbm, src_hbm, dst2_hbm, ew_hbm, out_hbm,
             norm_v, src_v, dst2_v, w_v,
             g0, g1, g2, g3, g4, sb0, sb1, sb2, sb3, acc,
             gm0, gm1, gm2, gm3, gm4, sm0, sm1, sm2, sm3):
    c = lax.axis_index("c")
    s = lax.axis_index("s")
    gbufs = ((g0, gm0), (g1, gm1), (g2, gm2), (g3, gm3), (g4, gm4))
    sbufs = ((sb0, sm0), (sb1, sm1), (sb2, sm2), (sb3, sm3))

    zero16 = jnp.zeros((16,), jnp.float32)

    @pl.loop(0, C)
    def _(r):
        for j in range(DH // 16):
            sb0[r, pl.ds(j * 16, 16)] = zero16

    for k in range(ZB):
        pltpu.sync_copy(sb0, acc.at[pl.ds(s * RPT + k * C, C)])

    pltpu.sync_copy(norm_hbm, norm_v)

    def start_gather(ci, buf, sem):
        pltpu.async_copy(hw_hbm.at[src_v.at[pl.ds(ci * C, C)]], buf, sem)

    for half in range(NH):
        pltpu.sync_copy(src_hbm.at[s, half], src_v)
        pltpu.sync_copy(dst2_hbm.at[s, half], dst2_v)
        pltpu.sync_copy(ew_hbm.at[s, half], w_v)

        @pl.loop(0, NCH)
        def _(ci):
            for g in range(C // 16):
                sl = pl.ds(ci * C + g * 16, 16)
                sv = src_v[sl]
                ns = plsc.load_gather(norm_v, [sv])
                nd = plsc.load_gather(norm_v, [dst2_v[ci, pl.ds(g * 16, 16)]])
                w_v[sl] = w_v[sl] * ns * nd
                src_v[sl] = sv * 2 + c

        for b, (gbuf, gsem) in enumerate(gbufs):
            start_gather(b, gbuf, gsem)

        @pl.loop(0, NCH, step=NBUF)
        def _(ci):
            for b in range(NBUF):
                gbuf, gsem = gbufs[b]
                sbuf, ssem = sbufs[SB_MAP[b]]
                cc = ci + b
                pltpu.make_async_copy(
                    hw_hbm.at[src_v.at[pl.ds(cc * C, C)]], gbuf, gsem).wait()

                @pl.when(cc >= SB_PREV[b])
                def _():
                    pltpu.make_async_copy(
                        sbuf, acc.at[dst2_v.at[0]], ssem).wait()

                @pl.loop(0, C, step=8)
                def _(e0):
                    for de in range(8):
                        e = e0 + de
                        bw = plsc.load_gather(
                            w_v, [jnp.full((16,), 0, jnp.int32) + cc * C + e])
                        for j in range(DH // 16):
                            sbuf[e, pl.ds(j * 16, 16)] = (
                                gbuf[e, pl.ds(j * 16, 16)] * bw)

                @pl.when(cc + NBUF < NCH)
                def _():
                    start_gather(cc + NBUF, gbuf, gsem)

                pltpu.async_copy(sbuf, acc.at[dst2_v.at[cc]], ssem, add=True)

        for sbuf, ssem in sbufs:
            pltpu.make_async_copy(
                sbuf, acc.at[dst2_v.at[0]], ssem).wait()

    plsc.subcore_barrier()

    pltpu.sync_copy(acc.at[pl.ds(s * RPT, RPT)],
                    out_hbm.at[c, pl.ds(s * RPT, RPT)])


@jax.jit
def kernel(h, norm, edge_weight, W, b, edge_index):
    src = edge_index[0].astype(jnp.int32).reshape(NS, NH, EH)
    dst2 = edge_index[1].astype(jnp.int32).reshape(NS, NH, NCH, C)
    ew = edge_weight.reshape(NS, NH, EH)

    hw = pl.pallas_call(
        _mm_body,
        grid=(10,),
        in_specs=[
            pl.BlockSpec((N // 10, D), lambda i: (i, 0)),
            pl.BlockSpec((D, D), lambda i: (0, 0)),
        ],
        out_specs=pl.BlockSpec((N // 10, D), lambda i: (i, 0)),
        out_shape=jax.ShapeDtypeStruct((N, D), jnp.float32),
    )(h, W)
    hw_half = hw.reshape(N * NC, DH)

    cp = pltpu.CompilerParams(needs_layout_passes=False,
                              use_tc_tiling_on_sc=False)
    mesh = plsc.VectorSubcoreMesh(core_axis_name="c", subcore_axis_name="s")
    sc_agg = pl.kernel(
        _sc_body,
        out_type=jax.ShapeDtypeStruct((NC, ACC_N, DH), jnp.float32),
        mesh=mesh,
        compiler_params=cp,
        scratch_types=(
            [
                pltpu.VMEM((N,), jnp.float32),
                pltpu.VMEM((EH,), jnp.int32),
                pltpu.VMEM((NCH, C), jnp.int32),
                pltpu.VMEM((EH,), jnp.float32),
            ]
            + [pltpu.VMEM((C, DH), jnp.float32)] * (NBUF + NSB)
            + [pltpu.VMEM_SHARED((ACC_N, DH), jnp.float32)]
            + [pltpu.SemaphoreType.DMA] * (NBUF + NSB)
        ),
    )
    partials = sc_agg(hw_half, norm, src, dst2, ew)

    out = pl.pallas_call(
        _combine_body,
        grid=(10,),
        in_specs=[
            pl.BlockSpec((NC, N // 10, DH), lambda i: (0, i, 0)),
            pl.BlockSpec((1, D), lambda i: (0, 0)),
        ],
        out_specs=pl.BlockSpec((N // 10, D), lambda i: (i, 0)),
        out_shape=jax.ShapeDtypeStruct((N, D), jnp.float32),
    )(partials, b.reshape(1, D))
    return out


# speedup vs baseline: 1.7227x; 1.7227x over previous
import dataclasses
import functools

import jax
import jax.numpy as jnp
from jax import lax
from jax.experimental import pallas as pl
from jax.experimental.pallas import tpu as pltpu
from jax.experimental.pallas import tpu_sc as plsc

N = 10000
E = 320000
D = 128
NC = 2
NS = 16
DH = D // NC
EP = E // NS
NH = 2
EH = EP // NH
C = 80
NCH = EH // C
ACC_N = 10240
RPT = ACC_N // NS
ZR = 64
ZB = RPT // ZR


def _mm_body(h_ref, n_ref, w_ref, o_ref):
    o_ref[...] = jnp.dot(h_ref[...] * n_ref[...], w_ref[...],
                         preferred_element_type=jnp.float32)


def _combine_body(p_ref, n_ref, b_ref, o_ref):
    o_ref[...] = (jnp.concatenate([p_ref[0], p_ref[1]], axis=1) * n_ref[...]
                  + b_ref[...])


def _sc_body(hw_hbm, src_hbm, dst2_hbm, ew_hbm, out_hbm,
             src_v, dst2_v, w_v, rows0_v, rows1_v, rows2_v, rows3_v,
             rows4_v, zb_v, acc, sem0, sem1, sem2, sem3, sem4,
             ssem0, ssem1, ssem2, ssem3, ssem4):
    c = lax.axis_index("c")
    s = lax.axis_index("s")
    bufs = ((rows0_v, sem0, ssem0), (rows1_v, sem1, ssem1),
            (rows2_v, sem2, ssem2), (rows3_v, sem3, ssem3),
            (rows4_v, sem4, ssem4))
    nbuf = len(bufs)

    zero16 = jnp.zeros((16,), jnp.float32)

    @pl.loop(0, ZR)
    def _(r):
        for j in range(DH // 16):
            zb_v[r, pl.ds(j * 16, 16)] = zero16

    for k in range(ZB):
        pltpu.sync_copy(zb_v, acc.at[pl.ds(s * RPT + k * ZR, ZR)])

    def start_gather(ci, buf, sem):
        pltpu.async_copy(hw_hbm.at[src_v.at[pl.ds(ci * C, C)]], buf, sem)

    for half in range(NH):
        pltpu.sync_copy(src_hbm.at[s, half], src_v)
        pltpu.sync_copy(dst2_hbm.at[s, half], dst2_v)
        pltpu.sync_copy(ew_hbm.at[s, half], w_v)

        @pl.loop(0, NCH)
        def _(ci):
            for g in range(C // 16):
                sl = pl.ds(ci * C + g * 16, 16)
                src_v[sl] = src_v[sl] * 2 + c

        for b, (buf, sem, _ssem) in enumerate(bufs):
            start_gather(b, buf, sem)

        @pl.loop(0, NCH, step=nbuf)
        def _(ci):
            for b, (buf, sem, ssem) in enumerate(bufs):
                cc = ci + b
                pltpu.make_async_copy(
                    hw_hbm.at[src_v.at[pl.ds(cc * C, C)]], buf, sem).wait()

                @pl.loop(0, C, step=8)
                def _(e0):
                    for de in range(8):
                        e = e0 + de
                        bw = plsc.load_gather(
                            w_v, [jnp.full((16,), 0, jnp.int32) + cc * C + e])
                        for j in range(DH // 16):
                            sl = (e, pl.ds(j * 16, 16))
                            buf[sl] = buf[sl] * bw

                pltpu.async_copy(buf, acc.at[dst2_v.at[cc]], ssem, add=True)

                @pl.when(cc + nbuf < NCH)
                def _():
                    pltpu.make_async_copy(
                        buf, acc.at[dst2_v.at[cc]], ssem).wait()
                    start_gather(cc + nbuf, buf, sem)

        for b, (buf, _sem, ssem) in enumerate(bufs):
            pltpu.make_async_copy(
                buf, acc.at[dst2_v.at[NCH - nbuf + b]], ssem).wait()

    plsc.subcore_barrier()

    pltpu.sync_copy(acc.at[pl.ds(s * RPT, RPT)],
                    out_hbm.at[c, pl.ds(s * RPT, RPT)])


@jax.jit
def kernel(h, norm, edge_weight, W, b, edge_index):
    src = edge_index[0].astype(jnp.int32).reshape(NS, NH, EH)
    dst2 = edge_index[1].astype(jnp.int32).reshape(NS, NH, NCH, C)
    ew = edge_weight.reshape(NS, NH, EH)

    hw = pl.pallas_call(
        _mm_body,
        grid=(10,),
        in_specs=[
            pl.BlockSpec((N // 10, D), lambda i: (i, 0)),
            pl.BlockSpec((N // 10, 1), lambda i: (i, 0)),
            pl.BlockSpec((D, D), lambda i: (0, 0)),
        ],
        out_specs=pl.BlockSpec((N // 10, D), lambda i: (i, 0)),
        out_shape=jax.ShapeDtypeStruct((N, D), jnp.float32),
    )(h, norm.reshape(N, 1), W)
    hw_half = hw.reshape(N * NC, DH)

    cp = pltpu.CompilerParams(needs_layout_passes=False,
                              use_tc_tiling_on_sc=False)
    mesh = plsc.VectorSubcoreMesh(core_axis_name="c", subcore_axis_name="s")
    sc_agg = pl.kernel(
        _sc_body,
        out_type=jax.ShapeDtypeStruct((NC, ACC_N, DH), jnp.float32),
        mesh=mesh,
        compiler_params=cp,
        scratch_types=[
            pltpu.VMEM((EH,), jnp.int32),
            pltpu.VMEM((NCH, C), jnp.int32),
            pltpu.VMEM((EH,), jnp.float32),
            pltpu.VMEM((C, DH), jnp.float32),
            pltpu.VMEM((C, DH), jnp.float32),
            pltpu.VMEM((C, DH), jnp.float32),
            pltpu.VMEM((C, DH), jnp.float32),
            pltpu.VMEM((C, DH), jnp.float32),
            pltpu.VMEM((ZR, DH), jnp.float32),
            pltpu.VMEM_SHARED((ACC_N, DH), jnp.float32),
            pltpu.SemaphoreType.DMA,
            pltpu.SemaphoreType.DMA,
            pltpu.SemaphoreType.DMA,
            pltpu.SemaphoreType.DMA,
            pltpu.SemaphoreType.DMA,
            pltpu.SemaphoreType.DMA,
            pltpu.SemaphoreType.DMA,
            pltpu.SemaphoreType.DMA,
            pltpu.SemaphoreType.DMA,
            pltpu.SemaphoreType.DMA,
        ],
    )
    partials = sc_agg(hw_half, src, dst2, ew)

    out = pl.pallas_call(
        _combine_body,
        grid=(10,),
        in_specs=[
            pl.BlockSpec((NC, N // 10, DH), lambda i: (0, i, 0)),
            pl.BlockSpec((N // 10, 1), lambda i: (i, 0)),
            pl.BlockSpec((1, D), lambda i: (0, 0)),
        ],
        out_specs=pl.BlockSpec((N // 10, D), lambda i: (i, 0)),
        out_shape=jax.ShapeDtypeStruct((N, D), jnp.float32),
    )(partials, norm.reshape(N, 1), b.reshape(1, D))
    return out


# speedup vs baseline: 2.0071x; 1.1651x over previous
import dataclasses
import functools

import jax
import jax.numpy as jnp
from jax import lax
from jax.experimental import pallas as pl
from jax.experimental.pallas import tpu as pltpu
from jax.experimental.pallas import tpu_sc as plsc

N = 10000
E = 320000
D = 128
NC = 2
NS = 16
DH = D // NC
EP = E // NS
NH = 2
EH = EP // NH
C = 80
NCH = EH // C
ACC_N = 10240
RPT = ACC_N // NS
ZR = 64
ZB = RPT // ZR


def _mm_body(h_ref, n_ref, w_ref, o_ref):
    o_ref[...] = jnp.dot(h_ref[...] * n_ref[...], w_ref[...],
                         preferred_element_type=jnp.float32)


def _combine_body(p_ref, n_ref, b_ref, o_ref):
    o_ref[...] = (jnp.concatenate([p_ref[0], p_ref[1]], axis=1) * n_ref[...]
                  + b_ref[...])


def _sc_body(hw_hbm, src_hbm, dst2_hbm, ew_hbm, out_hbm,
             src_v, dst2_v, w_v, rows0_v, rows1_v, rows2_v, rows3_v,
             rows4_v, zb_v, acc, sem0, sem1, sem2, sem3, sem4,
             ssem0, ssem1, ssem2, ssem3, ssem4):
    c = lax.axis_index("c")
    s = lax.axis_index("s")
    bufs = ((rows0_v, sem0, ssem0), (rows1_v, sem1, ssem1),
            (rows2_v, sem2, ssem2), (rows3_v, sem3, ssem3),
            (rows4_v, sem4, ssem4))
    nbuf = len(bufs)

    zero16 = jnp.zeros((16,), jnp.float32)

    @pl.loop(0, ZR)
    def _(r):
        for j in range(DH // 16):
            zb_v[r, pl.ds(j * 16, 16)] = zero16

    for k in range(ZB):
        pltpu.sync_copy(zb_v, acc.at[pl.ds(s * RPT + k * ZR, ZR)])

    def start_gather(ci, buf, sem):
        pltpu.async_copy(hw_hbm.at[src_v.at[pl.ds(ci * C, C)]], buf, sem)

    for half in range(NH):
        pltpu.sync_copy(src_hbm.at[s, half], src_v)
        pltpu.sync_copy(dst2_hbm.at[s, half], dst2_v)
        pltpu.sync_copy(ew_hbm.at[s, half], w_v)

        @pl.loop(0, NCH)
        def _(ci):
            for g in range(C // 16):
                sl = pl.ds(ci * C + g * 16, 16)
                src_v[sl] = src_v[sl] * 2 + c

        for b, (buf, sem, _ssem) in enumerate(bufs):
            start_gather(b, buf, sem)

        @pl.loop(0, NCH, step=nbuf)
        def _(ci):
            for b, (buf, sem, ssem) in enumerate(bufs):
                cc = ci + b
                pltpu.make_async_copy(
                    hw_hbm.at[src_v.at[pl.ds(cc * C, C)]], buf, sem).wait()

                @pl.loop(0, C, step=16)
                def _(e0):
                    w16 = w_v[pl.ds(cc * C + e0, 16)]
                    for de in range(16):
                        bw = w16[de]
                        for j in range(DH // 16):
                            sl = (e0 + de, pl.ds(j * 16, 16))
                            buf[sl] = buf[sl] * bw

                pltpu.async_copy(buf, acc.at[dst2_v.at[cc]], ssem, add=True)

                @pl.when(cc + nbuf < NCH)
                def _():
                    pltpu.make_async_copy(
                        buf, acc.at[dst2_v.at[cc]], ssem).wait()
                    start_gather(cc + nbuf, buf, sem)

        for b, (buf, _sem, ssem) in enumerate(bufs):
            pltpu.make_async_copy(
                buf, acc.at[dst2_v.at[NCH - nbuf + b]], ssem).wait()

    plsc.subcore_barrier()

    pltpu.sync_copy(acc.at[pl.ds(s * RPT, RPT)],
                    out_hbm.at[c, pl.ds(s * RPT, RPT)])


@jax.jit
def kernel(h, norm, edge_weight, W, b, edge_index):
    src = edge_index[0].astype(jnp.int32).reshape(NS, NH, EH)
    dst2 = edge_index[1].astype(jnp.int32).reshape(NS, NH, NCH, C)
    ew = edge_weight.reshape(NS, NH, EH)

    hw = pl.pallas_call(
        _mm_body,
        grid=(10,),
        in_specs=[
            pl.BlockSpec((N // 10, D), lambda i: (i, 0)),
            pl.BlockSpec((N // 10, 1), lambda i: (i, 0)),
            pl.BlockSpec((D, D), lambda i: (0, 0)),
        ],
        out_specs=pl.BlockSpec((N // 10, D), lambda i: (i, 0)),
        out_shape=jax.ShapeDtypeStruct((N, D), jnp.float32),
    )(h, norm.reshape(N, 1), W)
    hw_half = hw.reshape(N * NC, DH)

    cp = pltpu.CompilerParams(needs_layout_passes=False,
                              use_tc_tiling_on_sc=False)
    mesh = plsc.VectorSubcoreMesh(core_axis_name="c", subcore_axis_name="s")
    sc_agg = pl.kernel(
        _sc_body,
        out_type=jax.ShapeDtypeStruct((NC, ACC_N, DH), jnp.float32),
        mesh=mesh,
        compiler_params=cp,
        scratch_types=[
            pltpu.VMEM((EH,), jnp.int32),
            pltpu.VMEM((NCH, C), jnp.int32),
            pltpu.VMEM((EH,), jnp.float32),
            pltpu.VMEM((C, DH), jnp.float32),
            pltpu.VMEM((C, DH), jnp.float32),
            pltpu.VMEM((C, DH), jnp.float32),
            pltpu.VMEM((C, DH), jnp.float32),
            pltpu.VMEM((C, DH), jnp.float32),
            pltpu.VMEM((ZR, DH), jnp.float32),
            pltpu.VMEM_SHARED((ACC_N, DH), jnp.float32),
            pltpu.SemaphoreType.DMA,
            pltpu.SemaphoreType.DMA,
            pltpu.SemaphoreType.DMA,
            pltpu.SemaphoreType.DMA,
            pltpu.SemaphoreType.DMA,
            pltpu.SemaphoreType.DMA,
            pltpu.SemaphoreType.DMA,
            pltpu.SemaphoreType.DMA,
            pltpu.SemaphoreType.DMA,
            pltpu.SemaphoreType.DMA,
        ],
    )
    partials = sc_agg(hw_half, src, dst2, ew)

    out = pl.pallas_call(
        _combine_body,
        grid=(10,),
        in_specs=[
            pl.BlockSpec((NC, N // 10, DH), lambda i: (0, i, 0)),
            pl.BlockSpec((N // 10, 1), lambda i: (i, 0)),
            pl.BlockSpec((1, D), lambda i: (0, 0)),
        ],
        out_specs=pl.BlockSpec((N // 10, D), lambda i: (i, 0)),
        out_shape=jax.ShapeDtypeStruct((N, D), jnp.float32),
    )(partials, norm.reshape(N, 1), b.reshape(1, D))
    return out
